# Initial kernel scaffold; baseline (speedup 1.0000x reference)
import jax
import jax.numpy as jnp
from jax.experimental import pallas as pl


def kernel(X, edge_index, W1, b1, a1, a1b, W2, b2, a2, a2b, W3, b3):
    raise NotImplementedError("write your pallas kernel here")


# speedup vs baseline: 9.4874x; 9.4874x over previous
import functools

import numpy as np
import jax
import jax.numpy as jnp
from jax import lax
from jax.experimental import pallas as pl
from jax.experimental.pallas import tpu as pltpu
from jax.experimental.pallas import tpu_sc as plsc

N = 4096
D = 128
EDGES = 2 * 65536
NC = 2
NS = 16
NW = NC * NS
EPW = EDGES // NW
CH = EPW // 128
NROW = N // 128
BLK = 256
NB = N // BLK
THRESH = np.float32(0.6 ** 2)
_f32 = jnp.float32


def _scal_rows(aw, h, ab):
    s2 = lax.dot_general(aw, h, (((1,), (1,)), ((), ())),
                         preferred_element_type=_f32)
    row = lax.broadcasted_iota(jnp.int32, s2.shape, 0)
    return s2 + jnp.where(row == 0, ab, 0.0)


def _first_body(x_ref, w_ref, b_ref, aw_ref, ab_ref, h_ref, scal_ref):
    h = jnp.dot(x_ref[...], w_ref[...], preferred_element_type=_f32) + b_ref[...]
    h_ref[...] = h
    scal_ref[...] = _scal_rows(aw_ref[...], h, ab_ref[0, 0])


def _mm_first(x, w, b, aw, ab):
    k, dh = w.shape
    return pl.pallas_call(
        _first_body,
        grid=(NB,),
        in_specs=[
            pl.BlockSpec((BLK, k), lambda i: (i, 0)),
            pl.BlockSpec((k, dh), lambda i: (0, 0)),
            pl.BlockSpec((1, dh), lambda i: (0, 0)),
            pl.BlockSpec((2, dh), lambda i: (0, 0)),
            pl.BlockSpec((1, 1), lambda i: (0, 0)),
        ],
        out_specs=[
            pl.BlockSpec((BLK, dh), lambda i: (i, 0)),
            pl.BlockSpec((2, BLK), lambda i: (0, i)),
        ],
        out_shape=[jax.ShapeDtypeStruct((N, dh), _f32),
                   jax.ShapeDtypeStruct((2, N), _f32)],
    )(x, w, b, aw, ab)


def _mid_body(h_ref, hn_ref, dn_ref, w_ref, b_ref, aw_ref, ab_ref,
              h2_ref, scal_ref):
    hc = h_ref[...] + 0.5 * (hn_ref[0] + hn_ref[1] + dn_ref[...])
    hc = jnp.maximum(hc, 0.0)
    h2 = jnp.dot(hc, w_ref[...], preferred_element_type=_f32) + b_ref[...]
    h2_ref[...] = h2
    scal_ref[...] = _scal_rows(aw_ref[...], h2, ab_ref[0, 0])


def _mm_mid(h, hn, dn, w, b, aw, ab):
    k, dh = w.shape
    return pl.pallas_call(
        _mid_body,
        grid=(NB,),
        in_specs=[
            pl.BlockSpec((BLK, k), lambda i: (i, 0)),
            pl.BlockSpec((2, BLK, k), lambda i: (0, i, 0)),
            pl.BlockSpec((BLK, k), lambda i: (i, 0)),
            pl.BlockSpec((k, dh), lambda i: (0, 0)),
            pl.BlockSpec((1, dh), lambda i: (0, 0)),
            pl.BlockSpec((2, dh), lambda i: (0, 0)),
            pl.BlockSpec((1, 1), lambda i: (0, 0)),
        ],
        out_specs=[
            pl.BlockSpec((BLK, dh), lambda i: (i, 0)),
            pl.BlockSpec((2, BLK), lambda i: (0, i)),
        ],
        out_shape=[jax.ShapeDtypeStruct((N, dh), _f32),
                   jax.ShapeDtypeStruct((2, N), _f32)],
    )(h, hn, dn, w, b, aw, ab)


def _final_body(h_ref, hn_ref, dn_ref, w_ref, b_ref, o_ref):
    hc = h_ref[...] + 0.5 * (hn_ref[0] + hn_ref[1] + dn_ref[...])
    hc = jnp.maximum(hc, 0.0)
    o = jnp.dot(hc, w_ref[...], preferred_element_type=_f32) + b_ref[...]
    nrm = jnp.maximum(jnp.sqrt(jnp.sum(o * o, axis=1, keepdims=True)), 1e-12)
    o_ref[...] = o / nrm


def _mm_final(h, hn, dn, w, b):
    k, dh = w.shape
    return pl.pallas_call(
        _final_body,
        grid=(NB,),
        in_specs=[
            pl.BlockSpec((BLK, k), lambda i: (i, 0)),
            pl.BlockSpec((2, BLK, k), lambda i: (0, i, 0)),
            pl.BlockSpec((BLK, k), lambda i: (i, 0)),
            pl.BlockSpec((k, dh), lambda i: (0, 0)),
            pl.BlockSpec((1, dh), lambda i: (0, 0)),
        ],
        out_specs=pl.BlockSpec((BLK, dh), lambda i: (i, 0)),
        out_shape=jax.ShapeDtypeStruct((N, dh), _f32),
    )(h, hn, dn, w, b)


def _flash_body(hi_ref, hj_ref, aw_ref, ab_ref, acc_ref):
    hi = hi_ref[...]
    hj = hj_ref[...]
    aw = aw_ref[...]
    ones = jnp.ones((1, D), _f32)
    dn = (((1,), (1,)), ((), ()))
    si = lax.dot_general(hi, aw, dn, preferred_element_type=_f32)[:, 0:1]
    sj = lax.dot_general(aw, hj, dn, preferred_element_type=_f32)[1:2, :]
    ni = jnp.sqrt(lax.dot_general(hi * hi, ones, dn, preferred_element_type=_f32))
    nj = jnp.sqrt(lax.dot_general(ones, hj * hj, dn, preferred_element_type=_f32))
    s = lax.dot_general(hi, hj, dn, preferred_element_type=_f32)
    c = s / (jnp.maximum(ni, 1e-12) * jnp.maximum(nj, 1e-12))
    a = jnp.where(c > THRESH, c, 0.0)
    e = si + ab_ref[0, 0] + sj
    e = jnp.where(e >= 0, e, 0.01 * e)
    bmat = jnp.exp(a * e)

    @pl.when(pl.program_id(1) == 0)
    def _():
        acc_ref[...] = jnp.zeros_like(acc_ref)

    acc_ref[...] += jnp.dot(bmat, hj, preferred_element_type=_f32)


def _flash(h, aw, ab):
    return pl.pallas_call(
        _flash_body,
        grid=(NB, NB),
        in_specs=[
            pl.BlockSpec((BLK, D), lambda i, j: (i, 0)),
            pl.BlockSpec((BLK, D), lambda i, j: (j, 0)),
            pl.BlockSpec((2, D), lambda i, j: (0, 0)),
            pl.BlockSpec((1, 1), lambda i, j: (0, 0)),
        ],
        out_specs=pl.BlockSpec((BLK, D), lambda i, j: (i, 0)),
        out_shape=jax.ShapeDtypeStruct((N, D), _f32),
        compiler_params=pltpu.CompilerParams(
            dimension_semantics=("parallel", "arbitrary")),
    )(h, h, aw, ab)


def _sc_edge_a_body(ei_hbm, ej_hbm, scal_hbm, e_out, den_out,
                    ei_v, ej_v, si_v, sj_v, e_v, den_v, zrow_v, rowidx_v,
                    den_sh):
    c = lax.axis_index("c")
    s = lax.axis_index("s")
    w = s * NC + c
    pltpu.sync_copy(ei_hbm.at[w], ei_v)
    pltpu.sync_copy(ej_hbm.at[w], ej_v)
    pltpu.sync_copy(scal_hbm.at[0], si_v)
    pltpu.sync_copy(scal_hbm.at[1], sj_v)

    def zrow(r, carry):
        def zcol(g, carry2):
            den_v[r, pl.ds(g * 16, 16)] = jnp.zeros((16,), _f32)
            return 0
        return lax.fori_loop(0, 8, zcol, 0)
    lax.fori_loop(0, NROW, zrow, 0)

    for g in range(8):
        zrow_v[0, pl.ds(g * 16, 16)] = jnp.zeros((16,), _f32)
        zrow_v[1, pl.ds(g * 16, 16)] = jnp.zeros((16,), _f32)
    pltpu.sync_copy(zrow_v, den_sh.at[pl.ds(s * 2, 2)])

    def chunk(ch, carry):
        def grp(g, carry2):
            ii = ei_v[ch, pl.ds(g * 16, 16)]
            jj = ej_v[ch, pl.ds(g * 16, 16)]
            x = plsc.load_gather(si_v, [ii]) + plsc.load_gather(sj_v, [jj])
            ex = jnp.exp(jnp.where(x >= 0, x, 0.01 * x))
            e_v[ch, pl.ds(g * 16, 16)] = ex
            plsc.addupdate_scatter(
                den_v, [lax.shift_right_logical(ii, 7), ii & 127], ex)
            return 0
        return lax.fori_loop(0, 8, grp, 0)
    lax.fori_loop(0, CH, chunk, 0)

    pltpu.sync_copy(e_v, e_out.at[w])

    rowidx_v[pl.ds(0, 16)] = lax.iota(jnp.int32, 16)
    rowidx_v[pl.ds(16, 16)] = lax.iota(jnp.int32, 16) + 16
    plsc.subcore_barrier()
    pltpu.sync_copy(den_v, den_sh.at[rowidx_v], add=True)
    plsc.subcore_barrier()

    @pl.when(s == 0)
    def _():
        pltpu.sync_copy(den_sh, den_v)
        pltpu.sync_copy(den_v, den_out.at[c])


def _sc_edge_b_body(ei_hbm, ej_hbm, e_hbm, den_hbm, h_hbm, hn_out,
                    ei_v, ej_v, a_v, den0_v, den1_v, rows_v, stage_v, sem,
                    hn_sh):
    c = lax.axis_index("c")
    s = lax.axis_index("s")
    w = s * NC + c
    pltpu.sync_copy(ei_hbm.at[w], ei_v)
    pltpu.sync_copy(ej_hbm.at[w], ej_v)
    pltpu.sync_copy(e_hbm.at[w], a_v)
    pltpu.sync_copy(den_hbm.at[0], den0_v)
    pltpu.sync_copy(den_hbm.at[1], den1_v)

    def zrow(r, carry):
        def zcol(g, carry2):
            stage_v[r, pl.ds(g * 16, 16)] = jnp.zeros((16,), _f32)
            return 0
        return lax.fori_loop(0, D // 16, zcol, 0)
    lax.fori_loop(0, BLK, zrow, 0)
    pltpu.sync_copy(stage_v, hn_sh.at[pl.ds(s * BLK, BLK)])
    plsc.subcore_barrier()

    def chunk(ch, carry):
        cp = pltpu.async_copy(h_hbm.at[ej_v.at[ch]], rows_v, sem)
        def grp(g, carry2):
            ii = ei_v[ch, pl.ds(g * 16, 16)]
            hi = lax.shift_right_logical(ii, 7)
            lo = ii & 127
            dd = (plsc.load_gather(den0_v, [hi, lo])
                  + plsc.load_gather(den1_v, [hi, lo]))
            ex = a_v[ch, pl.ds(g * 16, 16)]
            a_v[ch, pl.ds(g * 16, 16)] = ex / jnp.maximum(dd, 1e-12)
            return 0
        lax.fori_loop(0, 8, grp, 0)
        cp.wait()
        def rowscale(g, carry2):
            av = a_v[ch, pl.ds(g * 16, 16)]
            base = g * 16
            for k in range(16):
                ar = av[k]
                for cc in range(D // 16):
                    sl = pl.ds(cc * 16, 16)
                    rows_v[base + k, sl] = rows_v[base + k, sl] * ar
            return 0
        lax.fori_loop(0, 8, rowscale, 0)
        pltpu.sync_copy(rows_v, hn_sh.at[ei_v.at[ch]], add=True)
        return 0
    lax.fori_loop(0, CH, chunk, 0)

    plsc.subcore_barrier()
    pltpu.sync_copy(hn_sh.at[pl.ds(s * BLK, BLK)], stage_v)
    pltpu.sync_copy(stage_v, hn_out.at[c, pl.ds(s * BLK, BLK)])


@functools.lru_cache(maxsize=None)
def _sc_kernels():
    mesh = plsc.VectorSubcoreMesh(core_axis_name="c", subcore_axis_name="s",
                                  num_cores=NC, num_subcores=NS)
    sc_a = pl.kernel(
        _sc_edge_a_body,
        out_type=[jax.ShapeDtypeStruct((NW, CH, 128), _f32),
                  jax.ShapeDtypeStruct((NC, NROW, 128), _f32)],
        mesh=mesh,
        scratch_types=[
            pltpu.VMEM((CH, 128), jnp.int32),
            pltpu.VMEM((CH, 128), jnp.int32),
            pltpu.VMEM((N,), _f32),
            pltpu.VMEM((N,), _f32),
            pltpu.VMEM((CH, 128), _f32),
            pltpu.VMEM((NROW, 128), _f32),
            pltpu.VMEM((2, 128), _f32),
            pltpu.VMEM((NW,), jnp.int32),
            pltpu.VMEM_SHARED((NROW, 128), _f32),
        ],
        compiler_params=pltpu.CompilerParams(needs_layout_passes=False),
    )
    sc_b = pl.kernel(
        _sc_edge_b_body,
        out_type=jax.ShapeDtypeStruct((NC, N, D), _f32),
        mesh=mesh,
        scratch_types=[
            pltpu.VMEM((CH, 128), jnp.int32),
            pltpu.VMEM((CH, 128), jnp.int32),
            pltpu.VMEM((CH, 128), _f32),
            pltpu.VMEM((NROW, 128), _f32),
            pltpu.VMEM((NROW, 128), _f32),
            pltpu.VMEM((128, D), _f32),
            pltpu.VMEM((BLK, D), _f32),
            pltpu.SemaphoreType.DMA,
            pltpu.VMEM_SHARED((N, D), _f32),
        ],
        compiler_params=pltpu.CompilerParams(needs_layout_passes=False),
    )
    return sc_a, sc_b


def kernel(X, edge_index, W1, b1, a1, a1b, W2, b2, a2, a2b, W3, b3):
    _sc_edge_a, _sc_edge_b = _sc_kernels()
    src = edge_index[0]
    dst = edge_index[1]
    ei = jnp.concatenate([src, dst]).reshape(NW, CH, 128)
    ej = jnp.concatenate([dst, src]).reshape(NW, CH, 128)

    h1, scal1 = _mm_first(X, W1, b1.reshape(1, -1), a1.reshape(2, -1),
                          a1b.reshape(1, 1))
    e1, den1 = _sc_edge_a(ei, ej, scal1)
    dense1 = _flash(h1, a1.reshape(2, -1), a1b.reshape(1, 1))
    hn1 = _sc_edge_b(ei, ej, e1, den1, h1)

    h2, scal2 = _mm_mid(h1, hn1, dense1, W2, b2.reshape(1, -1),
                        a2.reshape(2, -1), a2b.reshape(1, 1))
    e2, den2 = _sc_edge_a(ei, ej, scal2)
    dense2 = _flash(h2, a2.reshape(2, -1), a2b.reshape(1, 1))
    hn2 = _sc_edge_b(ei, ej, e2, den2, h2)

    return _mm_final(h2, hn2, dense2, W3, b3.reshape(1, -1))


# speedup vs baseline: 9.4911x; 1.0004x over previous
import functools

import numpy as np
import jax
import jax.numpy as jnp
from jax import lax
from jax.experimental import pallas as pl
from jax.experimental.pallas import tpu as pltpu
from jax.experimental.pallas import tpu_sc as plsc

N = 4096
D = 128
EDGES = 2 * 65536
NC = 2
NS = 16
NW = NC * NS
EPW = EDGES // NW
CH = EPW // 128
NROW = N // 128
BLK = 256
NB = N // BLK
THRESH = np.float32(0.6 ** 2)
_f32 = jnp.float32


def _scal_rows(aw, h, ab):
    s2 = lax.dot_general(aw, h, (((1,), (1,)), ((), ())),
                         preferred_element_type=_f32)
    row = lax.broadcasted_iota(jnp.int32, s2.shape, 0)
    return s2 + jnp.where(row == 0, ab, 0.0)


def _first_body(x_ref, w_ref, b_ref, aw_ref, ab_ref, h_ref, scal_ref):
    h = jnp.dot(x_ref[...], w_ref[...], preferred_element_type=_f32) + b_ref[...]
    h_ref[...] = h
    scal_ref[...] = _scal_rows(aw_ref[...], h, ab_ref[0, 0])


def _mm_first(x, w, b, aw, ab):
    k, dh = w.shape
    return pl.pallas_call(
        _first_body,
        grid=(NB,),
        in_specs=[
            pl.BlockSpec((BLK, k), lambda i: (i, 0)),
            pl.BlockSpec((k, dh), lambda i: (0, 0)),
            pl.BlockSpec((1, dh), lambda i: (0, 0)),
            pl.BlockSpec((2, dh), lambda i: (0, 0)),
            pl.BlockSpec((1, 1), lambda i: (0, 0)),
        ],
        out_specs=[
            pl.BlockSpec((BLK, dh), lambda i: (i, 0)),
            pl.BlockSpec((2, BLK), lambda i: (0, i)),
        ],
        out_shape=[jax.ShapeDtypeStruct((N, dh), _f32),
                   jax.ShapeDtypeStruct((2, N), _f32)],
    )(x, w, b, aw, ab)


def _mid_body(h_ref, hn_ref, dn_ref, w_ref, b_ref, aw_ref, ab_ref,
              h2_ref, scal_ref):
    hc = h_ref[...] + 0.5 * (hn_ref[0] + hn_ref[1] + dn_ref[...])
    hc = jnp.maximum(hc, 0.0)
    h2 = jnp.dot(hc, w_ref[...], preferred_element_type=_f32) + b_ref[...]
    h2_ref[...] = h2
    scal_ref[...] = _scal_rows(aw_ref[...], h2, ab_ref[0, 0])


def _mm_mid(h, hn, dn, w, b, aw, ab):
    k, dh = w.shape
    return pl.pallas_call(
        _mid_body,
        grid=(NB,),
        in_specs=[
            pl.BlockSpec((BLK, k), lambda i: (i, 0)),
            pl.BlockSpec((2, BLK, k), lambda i: (0, i, 0)),
            pl.BlockSpec((BLK, k), lambda i: (i, 0)),
            pl.BlockSpec((k, dh), lambda i: (0, 0)),
            pl.BlockSpec((1, dh), lambda i: (0, 0)),
            pl.BlockSpec((2, dh), lambda i: (0, 0)),
            pl.BlockSpec((1, 1), lambda i: (0, 0)),
        ],
        out_specs=[
            pl.BlockSpec((BLK, dh), lambda i: (i, 0)),
            pl.BlockSpec((2, BLK), lambda i: (0, i)),
        ],
        out_shape=[jax.ShapeDtypeStruct((N, dh), _f32),
                   jax.ShapeDtypeStruct((2, N), _f32)],
    )(h, hn, dn, w, b, aw, ab)


def _final_body(h_ref, hn_ref, dn_ref, w_ref, b_ref, o_ref):
    hc = h_ref[...] + 0.5 * (hn_ref[0] + hn_ref[1] + dn_ref[...])
    hc = jnp.maximum(hc, 0.0)
    o = jnp.dot(hc, w_ref[...], preferred_element_type=_f32) + b_ref[...]
    nrm = jnp.maximum(jnp.sqrt(jnp.sum(o * o, axis=1, keepdims=True)), 1e-12)
    o_ref[...] = o / nrm


def _mm_final(h, hn, dn, w, b):
    k, dh = w.shape
    return pl.pallas_call(
        _final_body,
        grid=(NB,),
        in_specs=[
            pl.BlockSpec((BLK, k), lambda i: (i, 0)),
            pl.BlockSpec((2, BLK, k), lambda i: (0, i, 0)),
            pl.BlockSpec((BLK, k), lambda i: (i, 0)),
            pl.BlockSpec((k, dh), lambda i: (0, 0)),
            pl.BlockSpec((1, dh), lambda i: (0, 0)),
        ],
        out_specs=pl.BlockSpec((BLK, dh), lambda i: (i, 0)),
        out_shape=jax.ShapeDtypeStruct((N, dh), _f32),
    )(h, hn, dn, w, b)


def _flash_body(hi_ref, hj_ref, aw_ref, ab_ref, acc_ref):
    hi = hi_ref[...]
    hj = hj_ref[...]
    aw = aw_ref[...]
    ones = jnp.ones((1, D), _f32)
    dn = (((1,), (1,)), ((), ()))
    si = lax.dot_general(hi, aw, dn, preferred_element_type=_f32)[:, 0:1]
    sj = lax.dot_general(aw, hj, dn, preferred_element_type=_f32)[1:2, :]
    ni = jnp.sqrt(lax.dot_general(hi * hi, ones, dn, preferred_element_type=_f32))
    nj = jnp.sqrt(lax.dot_general(ones, hj * hj, dn, preferred_element_type=_f32))
    s = lax.dot_general(hi, hj, dn, preferred_element_type=_f32)
    c = s / (jnp.maximum(ni, 1e-12) * jnp.maximum(nj, 1e-12))
    a = jnp.where(c > THRESH, c, 0.0)
    e = si + ab_ref[0, 0] + sj
    e = jnp.where(e >= 0, e, 0.01 * e)
    bmat = jnp.exp(a * e)

    @pl.when(pl.program_id(1) == 0)
    def _():
        acc_ref[...] = jnp.zeros_like(acc_ref)

    acc_ref[...] += jnp.dot(bmat, hj, preferred_element_type=_f32)


def _flash(h, aw, ab):
    return pl.pallas_call(
        _flash_body,
        grid=(NB, NB),
        in_specs=[
            pl.BlockSpec((BLK, D), lambda i, j: (i, 0)),
            pl.BlockSpec((BLK, D), lambda i, j: (j, 0)),
            pl.BlockSpec((2, D), lambda i, j: (0, 0)),
            pl.BlockSpec((1, 1), lambda i, j: (0, 0)),
        ],
        out_specs=pl.BlockSpec((BLK, D), lambda i, j: (i, 0)),
        out_shape=jax.ShapeDtypeStruct((N, D), _f32),
        compiler_params=pltpu.CompilerParams(
            dimension_semantics=("parallel", "arbitrary")),
    )(h, h, aw, ab)


def _sc_edge_a_body(ei_hbm, ej_hbm, scal_hbm, e_out, den_out,
                    ei_v, ej_v, si_v, sj_v, e_v, den_v, zrow_v, rowidx_v,
                    den_sh):
    c = lax.axis_index("c")
    s = lax.axis_index("s")
    w = s * NC + c
    pltpu.sync_copy(ei_hbm.at[w], ei_v)
    pltpu.sync_copy(ej_hbm.at[w], ej_v)
    pltpu.sync_copy(scal_hbm.at[0], si_v)
    pltpu.sync_copy(scal_hbm.at[1], sj_v)

    def zrow(r, carry):
        def zcol(g, carry2):
            den_v[r, pl.ds(g * 16, 16)] = jnp.zeros((16,), _f32)
            return 0
        return lax.fori_loop(0, 8, zcol, 0)
    lax.fori_loop(0, NROW, zrow, 0)

    for g in range(8):
        zrow_v[0, pl.ds(g * 16, 16)] = jnp.zeros((16,), _f32)
        zrow_v[1, pl.ds(g * 16, 16)] = jnp.zeros((16,), _f32)
    pltpu.sync_copy(zrow_v, den_sh.at[pl.ds(s * 2, 2)])

    def chunk(ch, carry):
        def grp(g, carry2):
            ii = ei_v[ch, pl.ds(g * 16, 16)]
            jj = ej_v[ch, pl.ds(g * 16, 16)]
            x = plsc.load_gather(si_v, [ii]) + plsc.load_gather(sj_v, [jj])
            ex = jnp.exp(jnp.where(x >= 0, x, 0.01 * x))
            e_v[ch, pl.ds(g * 16, 16)] = ex
            plsc.addupdate_scatter(
                den_v, [lax.shift_right_logical(ii, 7), ii & 127], ex)
            return 0
        return lax.fori_loop(0, 8, grp, 0)
    lax.fori_loop(0, CH, chunk, 0)

    pltpu.sync_copy(e_v, e_out.at[w])

    rowidx_v[pl.ds(0, 16)] = lax.iota(jnp.int32, 16)
    rowidx_v[pl.ds(16, 16)] = lax.iota(jnp.int32, 16) + 16
    plsc.subcore_barrier()
    pltpu.sync_copy(den_v, den_sh.at[rowidx_v], add=True)
    plsc.subcore_barrier()

    @pl.when(s == 0)
    def _():
        pltpu.sync_copy(den_sh, den_v)
        pltpu.sync_copy(den_v, den_out.at[c])


def _sc_edge_b_body(ei_hbm, ej_hbm, e_hbm, den_hbm, h_hbm, hn_out,
                    ei_v, ej_v, a_v, den0_v, den1_v, rows_v, stage_v, sem,
                    hn_sh):
    c = lax.axis_index("c")
    s = lax.axis_index("s")
    w = s * NC + c
    pltpu.sync_copy(ei_hbm.at[w], ei_v)
    pltpu.sync_copy(ej_hbm.at[w], ej_v)
    pltpu.sync_copy(e_hbm.at[w], a_v)
    pltpu.sync_copy(den_hbm.at[0], den0_v)
    pltpu.sync_copy(den_hbm.at[1], den1_v)

    def zrow(r, carry):
        def zcol(g, carry2):
            stage_v[r, pl.ds(g * 16, 16)] = jnp.zeros((16,), _f32)
            return 0
        return lax.fori_loop(0, D // 16, zcol, 0)
    lax.fori_loop(0, BLK, zrow, 0)
    pltpu.sync_copy(stage_v, hn_sh.at[pl.ds(s * BLK, BLK)])
    plsc.subcore_barrier()

    def chunk(ch, carry):
        cp = pltpu.async_copy(h_hbm.at[ej_v.at[ch]], rows_v, sem)
        def grp(g, carry2):
            ii = ei_v[ch, pl.ds(g * 16, 16)]
            hi = lax.shift_right_logical(ii, 7)
            lo = ii & 127
            dd = (plsc.load_gather(den0_v, [hi, lo])
                  + plsc.load_gather(den1_v, [hi, lo]))
            ex = a_v[ch, pl.ds(g * 16, 16)]
            a_v[ch, pl.ds(g * 16, 16)] = ex / jnp.maximum(dd, 1e-12)
            return 0
        lax.fori_loop(0, 8, grp, 0)
        cp.wait()
        def rowscale(g, carry2):
            av = a_v[ch, pl.ds(g * 16, 16)]
            base = g * 16
            for k in range(16):
                ar = av[k]
                for cc in range(D // 16):
                    sl = pl.ds(cc * 16, 16)
                    rows_v[base + k, sl] = rows_v[base + k, sl] * ar
            return 0
        lax.fori_loop(0, 8, rowscale, 0)
        pltpu.sync_copy(rows_v, hn_sh.at[ei_v.at[ch]], add=True)
        return 0
    lax.fori_loop(0, CH, chunk, 0)

    plsc.subcore_barrier()
    pltpu.sync_copy(hn_sh.at[pl.ds(s * BLK, BLK)], stage_v)
    pltpu.sync_copy(stage_v, hn_out.at[c, pl.ds(s * BLK, BLK)])


@functools.lru_cache(maxsize=None)
def _sc_kernels():
    mesh = plsc.VectorSubcoreMesh(core_axis_name="c", subcore_axis_name="s",
                                  num_cores=NC, num_subcores=NS)
    sc_a = pl.kernel(
        _sc_edge_a_body,
        out_type=[jax.ShapeDtypeStruct((NW, CH, 128), _f32),
                  jax.ShapeDtypeStruct((NC, NROW, 128), _f32)],
        mesh=mesh,
        scratch_types=[
            pltpu.VMEM((CH, 128), jnp.int32),
            pltpu.VMEM((CH, 128), jnp.int32),
            pltpu.VMEM((N,), _f32),
            pltpu.VMEM((N,), _f32),
            pltpu.VMEM((CH, 128), _f32),
            pltpu.VMEM((NROW, 128), _f32),
            pltpu.VMEM((2, 128), _f32),
            pltpu.VMEM((NW,), jnp.int32),
            pltpu.VMEM_SHARED((NROW, 128), _f32),
        ],
        compiler_params=pltpu.CompilerParams(needs_layout_passes=False),
    )
    sc_b = pl.kernel(
        _sc_edge_b_body,
        out_type=jax.ShapeDtypeStruct((NC, N, D), _f32),
        mesh=mesh,
        scratch_types=[
            pltpu.VMEM((CH, 128), jnp.int32),
            pltpu.VMEM((CH, 128), jnp.int32),
            pltpu.VMEM((CH, 128), _f32),
            pltpu.VMEM((NROW, 128), _f32),
            pltpu.VMEM((NROW, 128), _f32),
            pltpu.VMEM((128, D), _f32),
            pltpu.VMEM((BLK, D), _f32),
            pltpu.SemaphoreType.DMA,
            pltpu.VMEM_SHARED((N, D), _f32),
        ],
        compiler_params=pltpu.CompilerParams(needs_layout_passes=False),
    )
    return sc_a, sc_b


def kernel(X, edge_index, W1, b1, a1, a1b, W2, b2, a2, a2b, W3, b3):
    _sc_edge_a, _sc_edge_b = _sc_kernels()
    src = edge_index[0]
    dst = edge_index[1]
    ei = jnp.concatenate([src, dst]).reshape(NW, CH, 128)
    ej = jnp.concatenate([dst, src]).reshape(NW, CH, 128)

    h1, scal1 = _mm_first(X, W1, b1.reshape(1, -1), a1.reshape(2, -1),
                          a1b.reshape(1, 1))
    e1, den1 = _sc_edge_a(ei, ej, scal1)
    hn1 = _sc_edge_b(ei, ej, e1, den1, h1)
    dense1 = _flash(h1, a1.reshape(2, -1), a1b.reshape(1, 1))

    h2, scal2 = _mm_mid(h1, hn1, dense1, W2, b2.reshape(1, -1),
                        a2.reshape(2, -1), a2b.reshape(1, 1))
    e2, den2 = _sc_edge_a(ei, ej, scal2)
    hn2 = _sc_edge_b(ei, ej, e2, den2, h2)
    dense2 = _flash(h2, a2.reshape(2, -1), a2b.reshape(1, 1))

    return _mm_final(h2, hn2, dense2, W3, b3.reshape(1, -1))


# speedup vs baseline: 9.6877x; 1.0207x over previous
import functools

import numpy as np
import jax
import jax.numpy as jnp
from jax import lax
from jax.experimental import pallas as pl
from jax.experimental.pallas import tpu as pltpu
from jax.experimental.pallas import tpu_sc as plsc

N = 4096
D = 128
EDGES = 2 * 65536
NC = 2
NS = 16
NW = NC * NS
EPW = EDGES // NW
CH = EPW // 128
NROW = N // 128
BLK = 256
NB = N // BLK
THRESH = np.float32(0.6 ** 2)
_f32 = jnp.float32


def _scal_rows(aw, h, ab):
    s2 = lax.dot_general(aw, h, (((1,), (1,)), ((), ())),
                         preferred_element_type=_f32)
    row = lax.broadcasted_iota(jnp.int32, s2.shape, 0)
    return s2 + jnp.where(row == 0, ab, 0.0)


def _first_body(x_ref, w_ref, b_ref, aw_ref, ab_ref, h_ref, scal_ref):
    h = jnp.dot(x_ref[...], w_ref[...], preferred_element_type=_f32) + b_ref[...]
    h_ref[...] = h
    scal_ref[...] = _scal_rows(aw_ref[...], h, ab_ref[0, 0])


def _mm_first(x, w, b, aw, ab):
    k, dh = w.shape
    return pl.pallas_call(
        _first_body,
        grid=(NB,),
        in_specs=[
            pl.BlockSpec((BLK, k), lambda i: (i, 0)),
            pl.BlockSpec((k, dh), lambda i: (0, 0)),
            pl.BlockSpec((1, dh), lambda i: (0, 0)),
            pl.BlockSpec((2, dh), lambda i: (0, 0)),
            pl.BlockSpec((1, 1), lambda i: (0, 0)),
        ],
        out_specs=[
            pl.BlockSpec((BLK, dh), lambda i: (i, 0)),
            pl.BlockSpec((2, BLK), lambda i: (0, i)),
        ],
        out_shape=[jax.ShapeDtypeStruct((N, dh), _f32),
                   jax.ShapeDtypeStruct((2, N), _f32)],
    )(x, w, b, aw, ab)


def _mid_body(h_ref, hn_ref, dn_ref, w_ref, b_ref, aw_ref, ab_ref,
              h2_ref, scal_ref):
    hc = h_ref[...] + 0.5 * (hn_ref[0] + hn_ref[1] + dn_ref[...])
    hc = jnp.maximum(hc, 0.0)
    h2 = jnp.dot(hc, w_ref[...], preferred_element_type=_f32) + b_ref[...]
    h2_ref[...] = h2
    scal_ref[...] = _scal_rows(aw_ref[...], h2, ab_ref[0, 0])


def _mm_mid(h, hn, dn, w, b, aw, ab):
    k, dh = w.shape
    return pl.pallas_call(
        _mid_body,
        grid=(NB,),
        in_specs=[
            pl.BlockSpec((BLK, k), lambda i: (i, 0)),
            pl.BlockSpec((2, BLK, k), lambda i: (0, i, 0)),
            pl.BlockSpec((BLK, k), lambda i: (i, 0)),
            pl.BlockSpec((k, dh), lambda i: (0, 0)),
            pl.BlockSpec((1, dh), lambda i: (0, 0)),
            pl.BlockSpec((2, dh), lambda i: (0, 0)),
            pl.BlockSpec((1, 1), lambda i: (0, 0)),
        ],
        out_specs=[
            pl.BlockSpec((BLK, dh), lambda i: (i, 0)),
            pl.BlockSpec((2, BLK), lambda i: (0, i)),
        ],
        out_shape=[jax.ShapeDtypeStruct((N, dh), _f32),
                   jax.ShapeDtypeStruct((2, N), _f32)],
    )(h, hn, dn, w, b, aw, ab)


def _final_body(h_ref, hn_ref, dn_ref, w_ref, b_ref, o_ref):
    hc = h_ref[...] + 0.5 * (hn_ref[0] + hn_ref[1] + dn_ref[...])
    hc = jnp.maximum(hc, 0.0)
    o = jnp.dot(hc, w_ref[...], preferred_element_type=_f32) + b_ref[...]
    nrm = jnp.maximum(jnp.sqrt(jnp.sum(o * o, axis=1, keepdims=True)), 1e-12)
    o_ref[...] = o / nrm


def _mm_final(h, hn, dn, w, b):
    k, dh = w.shape
    return pl.pallas_call(
        _final_body,
        grid=(NB,),
        in_specs=[
            pl.BlockSpec((BLK, k), lambda i: (i, 0)),
            pl.BlockSpec((2, BLK, k), lambda i: (0, i, 0)),
            pl.BlockSpec((BLK, k), lambda i: (i, 0)),
            pl.BlockSpec((k, dh), lambda i: (0, 0)),
            pl.BlockSpec((1, dh), lambda i: (0, 0)),
        ],
        out_specs=pl.BlockSpec((BLK, dh), lambda i: (i, 0)),
        out_shape=jax.ShapeDtypeStruct((N, dh), _f32),
    )(h, hn, dn, w, b)


def _flash_body(hi_ref, hj_ref, aw_ref, ab_ref, acc_ref):
    hi = hi_ref[...]
    hj = hj_ref[...]
    hib = hi.astype(jnp.bfloat16)
    hjb = hj.astype(jnp.bfloat16)
    aw = aw_ref[...]
    ones = jnp.ones((1, D), _f32)
    dn = (((1,), (1,)), ((), ()))
    si = lax.dot_general(hi, aw, dn, preferred_element_type=_f32)[:, 0:1]
    sj = lax.dot_general(aw, hj, dn, preferred_element_type=_f32)[1:2, :]
    inv_ni = lax.rsqrt(jnp.maximum(
        lax.dot_general(hi * hi, ones, dn, preferred_element_type=_f32), 1e-24))
    inv_nj = lax.rsqrt(jnp.maximum(
        lax.dot_general(ones, hj * hj, dn, preferred_element_type=_f32), 1e-24))
    s = lax.dot_general(hib, hjb, dn, preferred_element_type=_f32)
    c = s * (inv_ni * inv_nj)
    a = jnp.where(c > THRESH, c, 0.0)
    e = si + ab_ref[0, 0] + sj
    e = jnp.where(e >= 0, e, 0.01 * e)
    bmat = jnp.exp(a * e)

    @pl.when(pl.program_id(1) == 0)
    def _():
        acc_ref[...] = jnp.zeros_like(acc_ref)

    acc_ref[...] += jnp.dot(bmat.astype(jnp.bfloat16), hjb,
                            preferred_element_type=_f32)


def _flash(h, aw, ab):
    return pl.pallas_call(
        _flash_body,
        grid=(NB, NB),
        in_specs=[
            pl.BlockSpec((BLK, D), lambda i, j: (i, 0)),
            pl.BlockSpec((BLK, D), lambda i, j: (j, 0)),
            pl.BlockSpec((2, D), lambda i, j: (0, 0)),
            pl.BlockSpec((1, 1), lambda i, j: (0, 0)),
        ],
        out_specs=pl.BlockSpec((BLK, D), lambda i, j: (i, 0)),
        out_shape=jax.ShapeDtypeStruct((N, D), _f32),
        compiler_params=pltpu.CompilerParams(
            dimension_semantics=("parallel", "arbitrary")),
    )(h, h, aw, ab)


def _sc_edge_a_body(ei_hbm, ej_hbm, scal_hbm, e_out, den_out,
                    ei_v, ej_v, si_v, sj_v, e_v, den_v, zrow_v, rowidx_v,
                    den_sh):
    c = lax.axis_index("c")
    s = lax.axis_index("s")
    w = s * NC + c
    pltpu.sync_copy(ei_hbm.at[w], ei_v)
    pltpu.sync_copy(ej_hbm.at[w], ej_v)
    pltpu.sync_copy(scal_hbm.at[0], si_v)
    pltpu.sync_copy(scal_hbm.at[1], sj_v)

    def zrow(r, carry):
        def zcol(g, carry2):
            den_v[r, pl.ds(g * 16, 16)] = jnp.zeros((16,), _f32)
            return 0
        return lax.fori_loop(0, 8, zcol, 0)
    lax.fori_loop(0, NROW, zrow, 0)

    for g in range(8):
        zrow_v[0, pl.ds(g * 16, 16)] = jnp.zeros((16,), _f32)
        zrow_v[1, pl.ds(g * 16, 16)] = jnp.zeros((16,), _f32)
    pltpu.sync_copy(zrow_v, den_sh.at[pl.ds(s * 2, 2)])

    def chunk(ch, carry):
        def grp(g, carry2):
            ii = ei_v[ch, pl.ds(g * 16, 16)]
            jj = ej_v[ch, pl.ds(g * 16, 16)]
            x = plsc.load_gather(si_v, [ii]) + plsc.load_gather(sj_v, [jj])
            ex = jnp.exp(jnp.where(x >= 0, x, 0.01 * x))
            e_v[ch, pl.ds(g * 16, 16)] = ex
            plsc.addupdate_scatter(
                den_v, [lax.shift_right_logical(ii, 7), ii & 127], ex)
            return 0
        return lax.fori_loop(0, 8, grp, 0)
    lax.fori_loop(0, CH, chunk, 0)

    pltpu.sync_copy(e_v, e_out.at[w])

    rowidx_v[pl.ds(0, 16)] = lax.iota(jnp.int32, 16)
    rowidx_v[pl.ds(16, 16)] = lax.iota(jnp.int32, 16) + 16
    plsc.subcore_barrier()
    pltpu.sync_copy(den_v, den_sh.at[rowidx_v], add=True)
    plsc.subcore_barrier()

    @pl.when(s == 0)
    def _():
        pltpu.sync_copy(den_sh, den_v)
        pltpu.sync_copy(den_v, den_out.at[c])


def _sc_edge_b_body(ei_hbm, ej_hbm, e_hbm, den_hbm, h_hbm, hn_out,
                    ei_v, ej_v, a_v, den0_v, den1_v, rows_v, stage_v, sem,
                    hn_sh):
    c = lax.axis_index("c")
    s = lax.axis_index("s")
    w = s * NC + c
    pltpu.sync_copy(ei_hbm.at[w], ei_v)
    pltpu.sync_copy(ej_hbm.at[w], ej_v)
    pltpu.sync_copy(e_hbm.at[w], a_v)
    pltpu.sync_copy(den_hbm.at[0], den0_v)
    pltpu.sync_copy(den_hbm.at[1], den1_v)

    def zrow(r, carry):
        def zcol(g, carry2):
            stage_v[r, pl.ds(g * 16, 16)] = jnp.zeros((16,), _f32)
            return 0
        return lax.fori_loop(0, D // 16, zcol, 0)
    lax.fori_loop(0, BLK, zrow, 0)
    pltpu.sync_copy(stage_v, hn_sh.at[pl.ds(s * BLK, BLK)])
    plsc.subcore_barrier()

    def chunk(ch, carry):
        cp = pltpu.async_copy(h_hbm.at[ej_v.at[ch]], rows_v, sem)
        def grp(g, carry2):
            ii = ei_v[ch, pl.ds(g * 16, 16)]
            hi = lax.shift_right_logical(ii, 7)
            lo = ii & 127
            dd = (plsc.load_gather(den0_v, [hi, lo])
                  + plsc.load_gather(den1_v, [hi, lo]))
            ex = a_v[ch, pl.ds(g * 16, 16)]
            a_v[ch, pl.ds(g * 16, 16)] = ex / jnp.maximum(dd, 1e-12)
            return 0
        lax.fori_loop(0, 8, grp, 0)
        cp.wait()
        def rowscale(g, carry2):
            av = a_v[ch, pl.ds(g * 16, 16)]
            base = g * 16
            for k in range(16):
                ar = av[k]
                for cc in range(D // 16):
                    sl = pl.ds(cc * 16, 16)
                    rows_v[base + k, sl] = rows_v[base + k, sl] * ar
            return 0
        lax.fori_loop(0, 8, rowscale, 0)
        pltpu.sync_copy(rows_v, hn_sh.at[ei_v.at[ch]], add=True)
        return 0
    lax.fori_loop(0, CH, chunk, 0)

    plsc.subcore_barrier()
    pltpu.sync_copy(hn_sh.at[pl.ds(s * BLK, BLK)], stage_v)
    pltpu.sync_copy(stage_v, hn_out.at[c, pl.ds(s * BLK, BLK)])


@functools.lru_cache(maxsize=None)
def _sc_kernels():
    mesh = plsc.VectorSubcoreMesh(core_axis_name="c", subcore_axis_name="s",
                                  num_cores=NC, num_subcores=NS)
    sc_a = pl.kernel(
        _sc_edge_a_body,
        out_type=[jax.ShapeDtypeStruct((NW, CH, 128), _f32),
                  jax.ShapeDtypeStruct((NC, NROW, 128), _f32)],
        mesh=mesh,
        scratch_types=[
            pltpu.VMEM((CH, 128), jnp.int32),
            pltpu.VMEM((CH, 128), jnp.int32),
            pltpu.VMEM((N,), _f32),
            pltpu.VMEM((N,), _f32),
            pltpu.VMEM((CH, 128), _f32),
            pltpu.VMEM((NROW, 128), _f32),
            pltpu.VMEM((2, 128), _f32),
            pltpu.VMEM((NW,), jnp.int32),
            pltpu.VMEM_SHARED((NROW, 128), _f32),
        ],
        compiler_params=pltpu.CompilerParams(needs_layout_passes=False),
    )
    sc_b = pl.kernel(
        _sc_edge_b_body,
        out_type=jax.ShapeDtypeStruct((NC, N, D), _f32),
        mesh=mesh,
        scratch_types=[
            pltpu.VMEM((CH, 128), jnp.int32),
            pltpu.VMEM((CH, 128), jnp.int32),
            pltpu.VMEM((CH, 128), _f32),
            pltpu.VMEM((NROW, 128), _f32),
            pltpu.VMEM((NROW, 128), _f32),
            pltpu.VMEM((128, D), _f32),
            pltpu.VMEM((BLK, D), _f32),
            pltpu.SemaphoreType.DMA,
            pltpu.VMEM_SHARED((N, D), _f32),
        ],
        compiler_params=pltpu.CompilerParams(needs_layout_passes=False),
    )
    return sc_a, sc_b


def kernel(X, edge_index, W1, b1, a1, a1b, W2, b2, a2, a2b, W3, b3):
    _sc_edge_a, _sc_edge_b = _sc_kernels()
    src = edge_index[0]
    dst = edge_index[1]
    ei = jnp.concatenate([src, dst]).reshape(NW, CH, 128)
    ej = jnp.concatenate([dst, src]).reshape(NW, CH, 128)

    h1, scal1 = _mm_first(X, W1, b1.reshape(1, -1), a1.reshape(2, -1),
                          a1b.reshape(1, 1))
    e1, den1 = _sc_edge_a(ei, ej, scal1)
    hn1 = _sc_edge_b(ei, ej, e1, den1, h1)
    dense1 = _flash(h1, a1.reshape(2, -1), a1b.reshape(1, 1))

    h2, scal2 = _mm_mid(h1, hn1, dense1, W2, b2.reshape(1, -1),
                        a2.reshape(2, -1), a2b.reshape(1, 1))
    e2, den2 = _sc_edge_a(ei, ej, scal2)
    hn2 = _sc_edge_b(ei, ej, e2, den2, h2)
    dense2 = _flash(h2, a2.reshape(2, -1), a2b.reshape(1, 1))

    return _mm_final(h2, hn2, dense2, W3, b3.reshape(1, -1))


# speedup vs baseline: 22.3614x; 2.3082x over previous
import functools

import numpy as np
import jax
import jax.numpy as jnp
from jax import lax
from jax.experimental import pallas as pl
from jax.experimental.pallas import tpu as pltpu
from jax.experimental.pallas import tpu_sc as plsc

N = 4096
D = 128
EDGES = 2 * 65536
NC = 2
NS = 16
NW = NC * NS
EPW = EDGES // NW
CH = EPW // 128
NROW = N // 128
BLK = 256
NB = N // BLK
BLKJ = 4096
THRESH = np.float32(0.6 ** 2)
_f32 = jnp.float32


def _scal_rows(aw, h, ab):
    s2 = lax.dot_general(aw, h, (((1,), (1,)), ((), ())),
                         preferred_element_type=_f32)
    row = lax.broadcasted_iota(jnp.int32, s2.shape, 0)
    return s2 + jnp.where(row == 0, ab, 0.0)


def _first_body(x_ref, w_ref, b_ref, aw_ref, ab_ref, h_ref, scal_ref):
    h = jnp.dot(x_ref[...], w_ref[...], preferred_element_type=_f32) + b_ref[...]
    h_ref[...] = h
    scal_ref[...] = _scal_rows(aw_ref[...], h, ab_ref[0, 0])


def _mm_first(x, w, b, aw, ab):
    k, dh = w.shape
    return pl.pallas_call(
        _first_body,
        grid=(NB,),
        in_specs=[
            pl.BlockSpec((BLK, k), lambda i: (i, 0)),
            pl.BlockSpec((k, dh), lambda i: (0, 0)),
            pl.BlockSpec((1, dh), lambda i: (0, 0)),
            pl.BlockSpec((2, dh), lambda i: (0, 0)),
            pl.BlockSpec((1, 1), lambda i: (0, 0)),
        ],
        out_specs=[
            pl.BlockSpec((BLK, dh), lambda i: (i, 0)),
            pl.BlockSpec((2, BLK), lambda i: (0, i)),
        ],
        out_shape=[jax.ShapeDtypeStruct((N, dh), _f32),
                   jax.ShapeDtypeStruct((2, N), _f32)],
    )(x, w, b, aw, ab)


def _mid_body(h_ref, hn_ref, dn_ref, w_ref, b_ref, aw_ref, ab_ref,
              h2_ref, scal_ref):
    hc = h_ref[...] + 0.5 * (hn_ref[0] + hn_ref[1] + dn_ref[...])
    hc = jnp.maximum(hc, 0.0)
    h2 = jnp.dot(hc, w_ref[...], preferred_element_type=_f32) + b_ref[...]
    h2_ref[...] = h2
    scal_ref[...] = _scal_rows(aw_ref[...], h2, ab_ref[0, 0])


def _mm_mid(h, hn, dn, w, b, aw, ab):
    k, dh = w.shape
    return pl.pallas_call(
        _mid_body,
        grid=(NB,),
        in_specs=[
            pl.BlockSpec((BLK, k), lambda i: (i, 0)),
            pl.BlockSpec((2, BLK, k), lambda i: (0, i, 0)),
            pl.BlockSpec((BLK, k), lambda i: (i, 0)),
            pl.BlockSpec((k, dh), lambda i: (0, 0)),
            pl.BlockSpec((1, dh), lambda i: (0, 0)),
            pl.BlockSpec((2, dh), lambda i: (0, 0)),
            pl.BlockSpec((1, 1), lambda i: (0, 0)),
        ],
        out_specs=[
            pl.BlockSpec((BLK, dh), lambda i: (i, 0)),
            pl.BlockSpec((2, BLK), lambda i: (0, i)),
        ],
        out_shape=[jax.ShapeDtypeStruct((N, dh), _f32),
                   jax.ShapeDtypeStruct((2, N), _f32)],
    )(h, hn, dn, w, b, aw, ab)


def _final_body(h_ref, hn_ref, dn_ref, w_ref, b_ref, o_ref):
    hc = h_ref[...] + 0.5 * (hn_ref[0] + hn_ref[1] + dn_ref[...])
    hc = jnp.maximum(hc, 0.0)
    o = jnp.dot(hc, w_ref[...], preferred_element_type=_f32) + b_ref[...]
    nrm = jnp.maximum(jnp.sqrt(jnp.sum(o * o, axis=1, keepdims=True)), 1e-12)
    o_ref[...] = o / nrm


def _mm_final(h, hn, dn, w, b):
    k, dh = w.shape
    return pl.pallas_call(
        _final_body,
        grid=(NB,),
        in_specs=[
            pl.BlockSpec((BLK, k), lambda i: (i, 0)),
            pl.BlockSpec((2, BLK, k), lambda i: (0, i, 0)),
            pl.BlockSpec((BLK, k), lambda i: (i, 0)),
            pl.BlockSpec((k, dh), lambda i: (0, 0)),
            pl.BlockSpec((1, dh), lambda i: (0, 0)),
        ],
        out_specs=pl.BlockSpec((BLK, dh), lambda i: (i, 0)),
        out_shape=jax.ShapeDtypeStruct((N, dh), _f32),
    )(h, hn, dn, w, b)


def _flash_body(hi_ref, hj_ref, aw_ref, ab_ref, acc_ref):
    hi = hi_ref[...]
    hj = hj_ref[...]
    hib = hi.astype(jnp.bfloat16)
    hjb = hj.astype(jnp.bfloat16)
    aw = aw_ref[...]
    ones = jnp.ones((1, D), _f32)
    dn = (((1,), (1,)), ((), ()))
    si = lax.dot_general(hi, aw, dn, preferred_element_type=_f32)[:, 0:1]
    sj = lax.dot_general(aw, hj, dn, preferred_element_type=_f32)[1:2, :]
    inv_ni = lax.rsqrt(jnp.maximum(
        lax.dot_general(hi * hi, ones, dn, preferred_element_type=_f32), 1e-24))
    inv_nj = lax.rsqrt(jnp.maximum(
        lax.dot_general(ones, hj * hj, dn, preferred_element_type=_f32), 1e-24))
    s = lax.dot_general(hib, hjb, dn, preferred_element_type=_f32)
    c = s * (inv_ni * inv_nj)
    a = jnp.where(c > THRESH, c, 0.0)
    e = si + ab_ref[0, 0] + sj
    e = jnp.where(e >= 0, e, 0.01 * e)
    bmat = jnp.exp(a * e)

    @pl.when(pl.program_id(1) == 0)
    def _():
        acc_ref[...] = jnp.zeros_like(acc_ref)

    acc_ref[...] += jnp.dot(bmat.astype(jnp.bfloat16), hjb,
                            preferred_element_type=_f32)


def _flash(h, aw, ab):
    return pl.pallas_call(
        _flash_body,
        grid=(NB, N // BLKJ),
        in_specs=[
            pl.BlockSpec((BLK, D), lambda i, j: (i, 0)),
            pl.BlockSpec((BLKJ, D), lambda i, j: (j, 0)),
            pl.BlockSpec((2, D), lambda i, j: (0, 0)),
            pl.BlockSpec((1, 1), lambda i, j: (0, 0)),
        ],
        out_specs=pl.BlockSpec((BLK, D), lambda i, j: (i, 0)),
        out_shape=jax.ShapeDtypeStruct((N, D), _f32),
        compiler_params=pltpu.CompilerParams(
            dimension_semantics=("parallel", "arbitrary")),
    )(h, h, aw, ab)


def _sc_layer_body(ei_hbm, ej_hbm, scal_hbm, h_hbm, hn_out,
                   ei0_v, ei1_v, ej0_v, ej1_v, si_v, sj_v, den_v,
                   zbuf_v, rows0_v, rows1_v, rowidx_v, sem0, sem1,
                   den_sh, hn_sh):
    c = lax.axis_index("c")
    s = lax.axis_index("s")
    pltpu.sync_copy(ei_hbm.at[s], ei0_v)
    pltpu.sync_copy(ei_hbm.at[s + NS], ei1_v)
    pltpu.sync_copy(ej_hbm.at[s], ej0_v)
    pltpu.sync_copy(ej_hbm.at[s + NS], ej1_v)
    pltpu.sync_copy(scal_hbm.at[0], si_v)
    pltpu.sync_copy(scal_hbm.at[1], sj_v)

    @pl.when(c == 0)
    def _():
        pltpu.async_copy(h_hbm.at[ej0_v.at[0]], rows0_v, sem0)
        pltpu.async_copy(h_hbm.at[ej0_v.at[1]], rows1_v, sem1)

    @pl.when(c == 1)
    def _():
        pltpu.async_copy(h_hbm.at[ej1_v.at[0]], rows0_v, sem0)
        pltpu.async_copy(h_hbm.at[ej1_v.at[1]], rows1_v, sem1)

    def zden(r, carry):
        def zcol(g, carry2):
            den_v[r, pl.ds(g * 16, 16)] = jnp.zeros((16,), _f32)
            return 0
        return lax.fori_loop(0, 8, zcol, 0)
    lax.fori_loop(0, NROW, zden, 0)

    def zbuf(r, carry):
        def zcol(g, carry2):
            zbuf_v[r, pl.ds(g * 16, 16)] = jnp.zeros((16,), _f32)
            return 0
        return lax.fori_loop(0, D // 16, zcol, 0)
    lax.fori_loop(0, 32, zbuf, 0)
    pltpu.sync_copy(zbuf_v.at[pl.ds(0, 2)], den_sh.at[pl.ds(s * 2, 2)])
    for piece in range(BLK // 32):
        pltpu.sync_copy(zbuf_v, hn_sh.at[pl.ds(s * BLK + piece * 32, 32)])

    def den_pass(ei_v, ej_v):
        def chunk(ch, carry):
            def grp(g, carry2):
                ii = ei_v[ch, pl.ds(g * 16, 16)]
                jj = ej_v[ch, pl.ds(g * 16, 16)]
                x = plsc.load_gather(si_v, [ii]) + plsc.load_gather(sj_v, [jj])
                ex = jnp.exp(jnp.where(x >= 0, x, 0.01 * x))
                plsc.addupdate_scatter(
                    den_v, [lax.shift_right_logical(ii, 7), ii & 127], ex)
                return 0
            return lax.fori_loop(0, 8, grp, 0)
        lax.fori_loop(0, CH, chunk, 0)
    den_pass(ei0_v, ej0_v)
    den_pass(ei1_v, ej1_v)

    rowidx_v[pl.ds(0, 16)] = lax.iota(jnp.int32, 16)
    rowidx_v[pl.ds(16, 16)] = lax.iota(jnp.int32, 16) + 16
    plsc.subcore_barrier()
    pltpu.sync_copy(den_v, den_sh.at[rowidx_v], add=True)
    plsc.subcore_barrier()
    pltpu.sync_copy(den_sh, den_v)

    def b_phase(ei_v, ej_v):
        def half(rows_v, sem, ch, nxt):
            pltpu.make_async_copy(h_hbm.at[ej_v.at[ch]], rows_v, sem).wait()
            def grpscale(g, carry2):
                ii = ei_v[ch, pl.ds(g * 16, 16)]
                jj = ej_v[ch, pl.ds(g * 16, 16)]
                x = (plsc.load_gather(si_v, [ii])
                     + plsc.load_gather(sj_v, [jj]))
                ex = jnp.exp(jnp.where(x >= 0, x, 0.01 * x))
                dd = plsc.load_gather(
                    den_v, [lax.shift_right_logical(ii, 7), ii & 127])
                av = ex / jnp.maximum(dd, 1e-12)
                base = g * 16
                for k in range(16):
                    ar = av[k]
                    for cc in range(D // 16):
                        sl = pl.ds(cc * 16, 16)
                        rows_v[base + k, sl] = rows_v[base + k, sl] * ar
                return 0
            lax.fori_loop(0, 8, grpscale, 0)
            pltpu.sync_copy(rows_v, hn_sh.at[ei_v.at[ch]], add=True)
            @pl.when(nxt < CH)
            def _():
                pltpu.async_copy(h_hbm.at[ej_v.at[nxt]], rows_v, sem)

        def chunk(t, carry):
            half(rows0_v, sem0, 2 * t, 2 * t + 2)
            half(rows1_v, sem1, 2 * t + 1, 2 * t + 3)
            return 0
        lax.fori_loop(0, CH // 2, chunk, 0)

    @pl.when(c == 0)
    def _():
        b_phase(ei0_v, ej0_v)

    @pl.when(c == 1)
    def _():
        b_phase(ei1_v, ej1_v)

    plsc.subcore_barrier()
    half_rows = BLK // 2
    pltpu.sync_copy(hn_sh.at[pl.ds(s * BLK, half_rows)], rows0_v)
    pltpu.sync_copy(rows0_v, hn_out.at[c, pl.ds(s * BLK, half_rows)])
    pltpu.sync_copy(hn_sh.at[pl.ds(s * BLK + half_rows, half_rows)], rows1_v)
    pltpu.sync_copy(rows1_v,
                    hn_out.at[c, pl.ds(s * BLK + half_rows, half_rows)])


@functools.lru_cache(maxsize=None)
def _sc_kernels():
    mesh = plsc.VectorSubcoreMesh(core_axis_name="c", subcore_axis_name="s",
                                  num_cores=NC, num_subcores=NS)
    sc_layer = pl.kernel(
        _sc_layer_body,
        out_type=jax.ShapeDtypeStruct((NC, N, D), _f32),
        mesh=mesh,
        scratch_types=[
            pltpu.VMEM((CH, 128), jnp.int32),
            pltpu.VMEM((CH, 128), jnp.int32),
            pltpu.VMEM((CH, 128), jnp.int32),
            pltpu.VMEM((CH, 128), jnp.int32),
            pltpu.VMEM((N,), _f32),
            pltpu.VMEM((N,), _f32),
            pltpu.VMEM((NROW, 128), _f32),
            pltpu.VMEM((32, D), _f32),
            pltpu.VMEM((128, D), _f32),
            pltpu.VMEM((128, D), _f32),
            pltpu.VMEM((NW,), jnp.int32),
            pltpu.SemaphoreType.DMA,
            pltpu.SemaphoreType.DMA,
            pltpu.VMEM_SHARED((NROW, 128), _f32),
            pltpu.VMEM_SHARED((N, D), _f32),
        ],
        compiler_params=pltpu.CompilerParams(needs_layout_passes=False),
    )
    return sc_layer


def kernel(X, edge_index, W1, b1, a1, a1b, W2, b2, a2, a2b, W3, b3):
    _sc_layer = _sc_kernels()
    src = edge_index[0]
    dst = edge_index[1]
    ei = jnp.concatenate([src, dst]).reshape(NW, CH, 128)
    ej = jnp.concatenate([dst, src]).reshape(NW, CH, 128)

    h1, scal1 = _mm_first(X, W1, b1.reshape(1, -1), a1.reshape(2, -1),
                          a1b.reshape(1, 1))
    hn1 = _sc_layer(ei, ej, scal1, h1)
    dense1 = _flash(h1, a1.reshape(2, -1), a1b.reshape(1, 1))

    h2, scal2 = _mm_mid(h1, hn1, dense1, W2, b2.reshape(1, -1),
                        a2.reshape(2, -1), a2b.reshape(1, 1))
    hn2 = _sc_layer(ei, ej, scal2, h2)
    dense2 = _flash(h2, a2.reshape(2, -1), a2b.reshape(1, 1))

    return _mm_final(h2, hn2, dense2, W3, b3.reshape(1, -1))
